# baseline (speedup 1.0000x reference)
import functools

import numpy as np
import jax
import jax.numpy as jnp
from jax import lax
from jax.experimental import pallas as pl
from jax.experimental.pallas import tpu as pltpu

_ALPHA = 1.0 / 100.0
_IGNITE_THR = 0.58
_WTA_INH = 0.85
_BIG = 0x3FFFFFFF


def _stage1_body(spk_hbm, n0_ref, rmax_ref, rm2_ref, gi1_ref, gi2_ref,
                 nmda_ref, xt_ref, xts_ref, sem, *, tt, w, b, nb, nt):
    bi = pl.program_id(0)
    ti = pl.program_id(1)
    it = ti

    @pl.when(it == 0)
    def _():
        nmda_ref[...] = n0_ref[0]

    c0 = jnp.float32(1.0 - _ALPHA)
    c1 = jnp.float32(_ALPHA)
    wta = jnp.float32(_WTA_INH)
    big = jnp.int32(_BIG)
    neg = jnp.float32(-jnp.inf)

    cur = bi * nt + ti
    k = lax.rem(cur, 2)

    def _copies(buf, tbi, tti):
        return [
            pltpu.make_async_copy(
                spk_hbm.at[pl.ds(tti * tt, tt), pl.ds(tbi * b + q * w, w)],
                xts_ref.at[buf, :, q, :],
                sem.at[buf, q])
            for q in range(8)
        ]

    @pl.when(cur == 0)
    def _():
        for c in _copies(0, bi, ti):
            c.start()

    nxt = cur + 1

    @pl.when(nxt < nb * nt)
    def _():
        nbi = nxt // nt
        nti = lax.rem(nxt, nt)
        for c in _copies(lax.rem(nxt, 2), nbi, nti):
            c.start()

    for c in _copies(k, bi, ti):
        c.wait()

    def body(t, nm):
        s = xts_ref[k, t]
        nm = c0 * nm + c1 * s
        xt_ref[pl.ds(t * 8, 8), :] = nm
        return nm

    nm_fin = lax.fori_loop(0, tt, body, nmda_ref[...])
    nmda_ref[...] = nm_fin

    r = tt * 8
    x = xt_ref[...]
    rmax = jnp.max(x, axis=1, keepdims=True)
    rm1 = wta * rmax
    scores = x * wta
    lane_r = lax.broadcasted_iota(jnp.int32, (r, w), 1)
    l1 = jnp.min(jnp.where(scores == rm1, lane_r, big), axis=1, keepdims=True)
    s2 = jnp.where(lane_r != l1, scores, neg)
    rm2 = jnp.max(s2, axis=1, keepdims=True)
    l2 = jnp.min(jnp.where(s2 == rm2, lane_r, big), axis=1, keepdims=True)

    sub = lax.broadcasted_iota(jnp.int32, (r, 1), 0) % 8
    base_r = pl.program_id(0) * (8 * w) + sub * w
    rmax_ref[0] = rmax
    rm2_ref[0] = rm2
    gi1_ref[0] = base_r + l1
    gi2_ref[0] = base_r + l2


def _stage2_body(rmax_ref, rm2_ref, gi1_ref, gi2_ref,
                 mask_ref, cov_ref, *, b2, cov_c):
    jb = pl.program_id(1)
    big = jnp.int32(_BIG)
    wta = jnp.float32(_WTA_INH)
    neg = jnp.float32(-jnp.inf)
    rmax = rmax_ref[...]
    v2 = rm2_ref[...]
    gi1 = gi1_ref[...]
    gi2 = gi2_ref[...]

    nmaxg = jnp.max(rmax, axis=1, keepdims=True)
    ign = nmaxg >= jnp.float32(_IGNITE_THR)

    v1 = rmax * wta
    gm1 = nmaxg * wta
    iw1 = jnp.min(jnp.where(v1 == gm1, gi1, big), axis=1, keepdims=True)

    v1x = jnp.where(gi1 == iw1, neg, v1)
    gm2 = jnp.maximum(jnp.max(v1x, axis=1, keepdims=True),
                      jnp.max(v2, axis=1, keepdims=True))
    c1b = jnp.min(jnp.where(v1x == gm2, gi1, big), axis=1, keepdims=True)
    c2b = jnp.min(jnp.where(v2 == gm2, gi2, big), axis=1, keepdims=True)
    iw2 = jnp.minimum(c1b, c2b)

    iw1m = jnp.where(ign, iw1, -1)
    iw2m = jnp.where(ign, iw2, -1)

    tt = rmax.shape[0]
    lanei = lax.broadcasted_iota(jnp.int32, (tt, b2), 1) + jb * b2
    m = (lanei == iw1m) | (lanei == iw2m)
    mask_ref[...] = m.astype(jnp.float32)

    @pl.when(jb == 0)
    def _():
        cov_ref[...] = jnp.where(ign, jnp.float32(cov_c), jnp.float32(0.0))


def kernel(spikes, nmda_state):
    t_dim, n_dim = spikes.shape
    b = 4096 if n_dim % 4096 == 0 else n_dim
    nb = n_dim // b
    w = b // 8
    nc = nb * 8
    tt = 256 if t_dim % 256 == 0 else t_dim
    nt = t_dim // tt

    n0 = nmda_state.reshape(nb, 8, w)

    cand_f = jax.ShapeDtypeStruct((nb, 8 * t_dim, 1), jnp.float32)
    cand_i = jax.ShapeDtypeStruct((nb, 8 * t_dim, 1), jnp.int32)

    s1 = pl.pallas_call(
        functools.partial(_stage1_body, tt=tt, w=w, b=b, nb=nb, nt=nt),
        grid=(nb, nt),
        in_specs=[
            pl.BlockSpec(memory_space=pl.ANY),
            pl.BlockSpec((1, 8, w), lambda bi, ti: (bi, 0, 0)),
        ],
        out_specs=[pl.BlockSpec((1, 8 * tt, 1), lambda bi, ti: (bi, ti, 0))] * 4,
        out_shape=[cand_f, cand_f, cand_i, cand_i],
        scratch_shapes=[
            pltpu.VMEM((8, w), jnp.float32),
            pltpu.VMEM((8 * tt, w), jnp.float32),
            pltpu.VMEM((2, tt, 8, w), jnp.float32),
            pltpu.SemaphoreType.DMA((2, 8)),
        ],
    )
    rmax, rm2, gi1, gi2 = s1(spikes, n0)

    def to_tc(x):
        return x.reshape(nb, t_dim, 8).transpose(1, 0, 2).reshape(t_dim, nc)

    rmax_t, rm2_t, gi1_t, gi2_t = map(to_tc, (rmax, rm2, gi1, gi2))

    b2 = 4096 if n_dim % 4096 == 0 else n_dim
    nb2 = n_dim // b2
    cov_c = float(np.float32(2.0) / np.float32(n_dim))

    s2 = pl.pallas_call(
        functools.partial(_stage2_body, b2=b2, cov_c=cov_c),
        grid=(nt, nb2),
        in_specs=[pl.BlockSpec((tt, nc), lambda ti, jb: (ti, 0))] * 4,
        out_specs=[
            pl.BlockSpec((tt, b2), lambda ti, jb: (ti, jb)),
            pl.BlockSpec((tt, 1), lambda ti, jb: (ti, 0)),
        ],
        out_shape=[
            jax.ShapeDtypeStruct((t_dim, n_dim), jnp.float32),
            jax.ShapeDtypeStruct((t_dim, 1), jnp.float32),
        ],
    )
    mask, cov = s2(rmax_t, rm2_t, gi1_t, gi2_t)
    return mask, cov.reshape(t_dim)


# speedup vs baseline: 1.1591x; 1.1591x over previous
import functools

import numpy as np
import jax
import jax.numpy as jnp
from jax import lax
from jax.experimental import pallas as pl
from jax.experimental.pallas import tpu as pltpu

_ALPHA = 1.0 / 100.0
_IGNITE_THR = 0.58
_WTA_INH = 0.85
_BIG = 0x3FFFFFFF


def _stage1_body(spk_hbm, n0_ref, rmax_ref, rm2_ref, gi1_ref, gi2_ref,
                 nmda_ref, xt_ref, xts_ref, sem, *, tt, w, b, nb, nt):
    bi = pl.program_id(0)
    ti = pl.program_id(1)
    it = ti

    @pl.when(it == 0)
    def _():
        nmda_ref[...] = n0_ref[0]

    c0 = jnp.float32(1.0 - _ALPHA)
    c1 = jnp.float32(_ALPHA)
    wta = jnp.float32(_WTA_INH)
    big = jnp.int32(_BIG)
    neg = jnp.float32(-jnp.inf)

    cur = bi * nt + ti
    k = lax.rem(cur, 2)

    def _copies(buf, tbi, tti):
        return [
            pltpu.make_async_copy(
                spk_hbm.at[pl.ds(tti * tt, tt), pl.ds(tbi * b + q * w, w)],
                xts_ref.at[buf, :, q, :],
                sem.at[buf, q])
            for q in range(8)
        ]

    @pl.when(cur == 0)
    def _():
        for c in _copies(0, bi, ti):
            c.start()

    nxt = cur + 1

    @pl.when(nxt < nb * nt)
    def _():
        nbi = nxt // nt
        nti = lax.rem(nxt, nt)
        for c in _copies(lax.rem(nxt, 2), nbi, nti):
            c.start()

    for c in _copies(k, bi, ti):
        c.wait()

    def body(t, nm):
        s = xts_ref[k, t]
        nm = c0 * nm + c1 * s
        xt_ref[pl.ds(t * 8, 8), :] = nm
        return nm

    nm_fin = lax.fori_loop(0, tt, body, nmda_ref[...], unroll=8)
    nmda_ref[...] = nm_fin

    r = tt * 8
    x = xt_ref[...]
    rmax = jnp.max(x, axis=1, keepdims=True)
    rm1 = wta * rmax
    scores = x * wta
    lane_r = lax.broadcasted_iota(jnp.int32, (r, w), 1)
    l1 = jnp.min(jnp.where(scores == rm1, lane_r, big), axis=1, keepdims=True)
    s2 = jnp.where(lane_r != l1, scores, neg)
    rm2 = jnp.max(s2, axis=1, keepdims=True)
    l2 = jnp.min(jnp.where(s2 == rm2, lane_r, big), axis=1, keepdims=True)

    sub = lax.broadcasted_iota(jnp.int32, (r, 1), 0) % 8
    base_r = pl.program_id(0) * (8 * w) + sub * w
    rmax_ref[0] = rmax
    rm2_ref[0] = rm2
    gi1_ref[0] = base_r + l1
    gi2_ref[0] = base_r + l2


def _stage2_body(rmax_ref, rm2_ref, gi1_ref, gi2_ref,
                 mask_ref, cov_ref, *, b2, cov_c):
    jb = pl.program_id(1)
    big = jnp.int32(_BIG)
    wta = jnp.float32(_WTA_INH)
    neg = jnp.float32(-jnp.inf)
    rmax = rmax_ref[...]
    v2 = rm2_ref[...]
    gi1 = gi1_ref[...]
    gi2 = gi2_ref[...]

    nmaxg = jnp.max(rmax, axis=1, keepdims=True)
    ign = nmaxg >= jnp.float32(_IGNITE_THR)

    v1 = rmax * wta
    gm1 = nmaxg * wta
    iw1 = jnp.min(jnp.where(v1 == gm1, gi1, big), axis=1, keepdims=True)

    v1x = jnp.where(gi1 == iw1, neg, v1)
    gm2 = jnp.maximum(jnp.max(v1x, axis=1, keepdims=True),
                      jnp.max(v2, axis=1, keepdims=True))
    c1b = jnp.min(jnp.where(v1x == gm2, gi1, big), axis=1, keepdims=True)
    c2b = jnp.min(jnp.where(v2 == gm2, gi2, big), axis=1, keepdims=True)
    iw2 = jnp.minimum(c1b, c2b)

    iw1m = jnp.where(ign, iw1, -1)
    iw2m = jnp.where(ign, iw2, -1)

    tt = rmax.shape[0]
    lanei = lax.broadcasted_iota(jnp.int32, (tt, b2), 1) + jb * b2
    m = (lanei == iw1m) | (lanei == iw2m)
    mask_ref[...] = m.astype(jnp.float32)

    @pl.when(jb == 0)
    def _():
        cov_ref[...] = jnp.where(ign, jnp.float32(cov_c), jnp.float32(0.0))


def kernel(spikes, nmda_state):
    t_dim, n_dim = spikes.shape
    b = 4096 if n_dim % 4096 == 0 else n_dim
    nb = n_dim // b
    w = b // 8
    nc = nb * 8
    tt = 256 if t_dim % 256 == 0 else t_dim
    nt = t_dim // tt

    n0 = nmda_state.reshape(nb, 8, w)

    cand_f = jax.ShapeDtypeStruct((nb, 8 * t_dim, 1), jnp.float32)
    cand_i = jax.ShapeDtypeStruct((nb, 8 * t_dim, 1), jnp.int32)

    s1 = pl.pallas_call(
        functools.partial(_stage1_body, tt=tt, w=w, b=b, nb=nb, nt=nt),
        grid=(nb, nt),
        in_specs=[
            pl.BlockSpec(memory_space=pl.ANY),
            pl.BlockSpec((1, 8, w), lambda bi, ti: (bi, 0, 0)),
        ],
        out_specs=[pl.BlockSpec((1, 8 * tt, 1), lambda bi, ti: (bi, ti, 0))] * 4,
        out_shape=[cand_f, cand_f, cand_i, cand_i],
        scratch_shapes=[
            pltpu.VMEM((8, w), jnp.float32),
            pltpu.VMEM((8 * tt, w), jnp.float32),
            pltpu.VMEM((2, tt, 8, w), jnp.float32),
            pltpu.SemaphoreType.DMA((2, 8)),
        ],
    )
    rmax, rm2, gi1, gi2 = s1(spikes, n0)

    def to_tc(x):
        return x.reshape(nb, t_dim, 8).transpose(1, 0, 2).reshape(t_dim, nc)

    rmax_t, rm2_t, gi1_t, gi2_t = map(to_tc, (rmax, rm2, gi1, gi2))

    b2 = 4096 if n_dim % 4096 == 0 else n_dim
    nb2 = n_dim // b2
    cov_c = float(np.float32(2.0) / np.float32(n_dim))

    s2 = pl.pallas_call(
        functools.partial(_stage2_body, b2=b2, cov_c=cov_c),
        grid=(nt, nb2),
        in_specs=[pl.BlockSpec((tt, nc), lambda ti, jb: (ti, 0))] * 4,
        out_specs=[
            pl.BlockSpec((tt, b2), lambda ti, jb: (ti, jb)),
            pl.BlockSpec((tt, 1), lambda ti, jb: (ti, 0)),
        ],
        out_shape=[
            jax.ShapeDtypeStruct((t_dim, n_dim), jnp.float32),
            jax.ShapeDtypeStruct((t_dim, 1), jnp.float32),
        ],
    )
    mask, cov = s2(rmax_t, rm2_t, gi1_t, gi2_t)
    return mask, cov.reshape(t_dim)


# speedup vs baseline: 1.1615x; 1.0021x over previous
import functools

import numpy as np
import jax
import jax.numpy as jnp
from jax import lax
from jax.experimental import pallas as pl
from jax.experimental.pallas import tpu as pltpu

_ALPHA = 1.0 / 100.0
_IGNITE_THR = 0.58
_WTA_INH = 0.85
_BIG = 0x3FFFFFFF


def _stage1_body(spk_hbm, n0_ref, rmax_ref, rm2_ref, gi1_ref, gi2_ref,
                 nmda_ref, xt_ref, xts_ref, sem, *, tt, w, b, nb, nt):
    bi = pl.program_id(0)
    ti = pl.program_id(1)
    it = ti

    @pl.when(it == 0)
    def _():
        nmda_ref[...] = n0_ref[0]

    c0 = jnp.float32(1.0 - _ALPHA)
    c1 = jnp.float32(_ALPHA)
    wta = jnp.float32(_WTA_INH)
    big = jnp.int32(_BIG)
    neg = jnp.float32(-jnp.inf)

    cur = bi * nt + ti
    k = lax.rem(cur, 2)

    def _copies(buf, tbi, tti):
        return [
            pltpu.make_async_copy(
                spk_hbm.at[pl.ds(tti * tt, tt), pl.ds(tbi * b + q * w, w)],
                xts_ref.at[buf, :, q, :],
                sem.at[buf, q])
            for q in range(8)
        ]

    @pl.when(cur == 0)
    def _():
        for c in _copies(0, bi, ti):
            c.start()

    nxt = cur + 1

    @pl.when(nxt < nb * nt)
    def _():
        nbi = nxt // nt
        nti = lax.rem(nxt, nt)
        for c in _copies(lax.rem(nxt, 2), nbi, nti):
            c.start()

    for c in _copies(k, bi, ti):
        c.wait()

    def body(t, nm):
        s = xts_ref[k, t]
        nm = c0 * nm + c1 * s
        xt_ref[pl.ds(t * 8, 8), :] = nm
        return nm

    nm_fin = lax.fori_loop(0, tt, body, nmda_ref[...], unroll=16)
    nmda_ref[...] = nm_fin

    r = tt * 8
    x = xt_ref[...]
    rmax = jnp.max(x, axis=1, keepdims=True)
    rm1 = wta * rmax
    scores = x * wta
    lane_r = lax.broadcasted_iota(jnp.int32, (r, w), 1)
    l1 = jnp.min(jnp.where(scores == rm1, lane_r, big), axis=1, keepdims=True)
    s2 = jnp.where(lane_r != l1, scores, neg)
    rm2 = jnp.max(s2, axis=1, keepdims=True)
    l2 = jnp.min(jnp.where(s2 == rm2, lane_r, big), axis=1, keepdims=True)

    sub = lax.broadcasted_iota(jnp.int32, (r, 1), 0) % 8
    base_r = pl.program_id(0) * (8 * w) + sub * w
    rmax_ref[0] = rmax
    rm2_ref[0] = rm2
    gi1_ref[0] = base_r + l1
    gi2_ref[0] = base_r + l2


def _stage2_body(rmax_ref, rm2_ref, gi1_ref, gi2_ref,
                 mask_ref, cov_ref, *, b2, cov_c):
    jb = pl.program_id(1)
    big = jnp.int32(_BIG)
    wta = jnp.float32(_WTA_INH)
    neg = jnp.float32(-jnp.inf)
    rmax = rmax_ref[...]
    v2 = rm2_ref[...]
    gi1 = gi1_ref[...]
    gi2 = gi2_ref[...]

    nmaxg = jnp.max(rmax, axis=1, keepdims=True)
    ign = nmaxg >= jnp.float32(_IGNITE_THR)

    v1 = rmax * wta
    gm1 = nmaxg * wta
    iw1 = jnp.min(jnp.where(v1 == gm1, gi1, big), axis=1, keepdims=True)

    v1x = jnp.where(gi1 == iw1, neg, v1)
    gm2 = jnp.maximum(jnp.max(v1x, axis=1, keepdims=True),
                      jnp.max(v2, axis=1, keepdims=True))
    c1b = jnp.min(jnp.where(v1x == gm2, gi1, big), axis=1, keepdims=True)
    c2b = jnp.min(jnp.where(v2 == gm2, gi2, big), axis=1, keepdims=True)
    iw2 = jnp.minimum(c1b, c2b)

    iw1m = jnp.where(ign, iw1, -1)
    iw2m = jnp.where(ign, iw2, -1)

    tt = rmax.shape[0]
    lanei = lax.broadcasted_iota(jnp.int32, (tt, b2), 1) + jb * b2
    m = (lanei == iw1m) | (lanei == iw2m)
    mask_ref[...] = m.astype(jnp.float32)

    @pl.when(jb == 0)
    def _():
        cov_ref[...] = jnp.where(ign, jnp.float32(cov_c), jnp.float32(0.0))


def kernel(spikes, nmda_state):
    t_dim, n_dim = spikes.shape
    b = 4096 if n_dim % 4096 == 0 else n_dim
    nb = n_dim // b
    w = b // 8
    nc = nb * 8
    tt = 256 if t_dim % 256 == 0 else t_dim
    nt = t_dim // tt

    n0 = nmda_state.reshape(nb, 8, w)

    cand_f = jax.ShapeDtypeStruct((nb, 8 * t_dim, 1), jnp.float32)
    cand_i = jax.ShapeDtypeStruct((nb, 8 * t_dim, 1), jnp.int32)

    s1 = pl.pallas_call(
        functools.partial(_stage1_body, tt=tt, w=w, b=b, nb=nb, nt=nt),
        grid=(nb, nt),
        in_specs=[
            pl.BlockSpec(memory_space=pl.ANY),
            pl.BlockSpec((1, 8, w), lambda bi, ti: (bi, 0, 0)),
        ],
        out_specs=[pl.BlockSpec((1, 8 * tt, 1), lambda bi, ti: (bi, ti, 0))] * 4,
        out_shape=[cand_f, cand_f, cand_i, cand_i],
        scratch_shapes=[
            pltpu.VMEM((8, w), jnp.float32),
            pltpu.VMEM((8 * tt, w), jnp.float32),
            pltpu.VMEM((2, tt, 8, w), jnp.float32),
            pltpu.SemaphoreType.DMA((2, 8)),
        ],
    )
    rmax, rm2, gi1, gi2 = s1(spikes, n0)

    def to_tc(x):
        return x.reshape(nb, t_dim, 8).transpose(1, 0, 2).reshape(t_dim, nc)

    rmax_t, rm2_t, gi1_t, gi2_t = map(to_tc, (rmax, rm2, gi1, gi2))

    b2 = 4096 if n_dim % 4096 == 0 else n_dim
    nb2 = n_dim // b2
    cov_c = float(np.float32(2.0) / np.float32(n_dim))

    s2 = pl.pallas_call(
        functools.partial(_stage2_body, b2=b2, cov_c=cov_c),
        grid=(nt, nb2),
        in_specs=[pl.BlockSpec((tt, nc), lambda ti, jb: (ti, 0))] * 4,
        out_specs=[
            pl.BlockSpec((tt, b2), lambda ti, jb: (ti, jb)),
            pl.BlockSpec((tt, 1), lambda ti, jb: (ti, 0)),
        ],
        out_shape=[
            jax.ShapeDtypeStruct((t_dim, n_dim), jnp.float32),
            jax.ShapeDtypeStruct((t_dim, 1), jnp.float32),
        ],
    )
    mask, cov = s2(rmax_t, rm2_t, gi1_t, gi2_t)
    return mask, cov.reshape(t_dim)


# speedup vs baseline: 1.1866x; 1.0216x over previous
import functools

import numpy as np
import jax
import jax.numpy as jnp
from jax import lax
from jax.experimental import pallas as pl
from jax.experimental.pallas import tpu as pltpu

_ALPHA = 1.0 / 100.0
_IGNITE_THR = 0.58
_WTA_INH = 0.85
_BIG = 0x3FFFFFFF


def _stage1_body(spk_hbm, n0_ref, rmax_ref, rm2_ref, gi1_ref, gi2_ref,
                 nmda_ref, xt_ref, xts_ref, sem, *, tt, w, b, nb, nt):
    bi = pl.program_id(0)
    ti = pl.program_id(1)
    it = ti

    @pl.when(it == 0)
    def _():
        nmda_ref[...] = n0_ref[0]

    c0 = jnp.float32(1.0 - _ALPHA)
    c1 = jnp.float32(_ALPHA)
    wta = jnp.float32(_WTA_INH)
    big = jnp.int32(_BIG)
    neg = jnp.float32(-jnp.inf)

    cur = bi * nt + ti
    k = lax.rem(cur, 2)

    def _copies(buf, tbi, tti):
        return [
            pltpu.make_async_copy(
                spk_hbm.at[pl.ds(tti * tt, tt), pl.ds(tbi * b + q * w, w)],
                xts_ref.at[buf, :, q, :],
                sem.at[buf, q])
            for q in range(8)
        ]

    @pl.when(cur == 0)
    def _():
        for c in _copies(0, bi, ti):
            c.start()

    nxt = cur + 1

    @pl.when(nxt < nb * nt)
    def _():
        nbi = nxt // nt
        nti = lax.rem(nxt, nt)
        for c in _copies(lax.rem(nxt, 2), nbi, nti):
            c.start()

    for c in _copies(k, bi, ti):
        c.wait()

    def body(t, nm):
        s = xts_ref[k, t]
        nm = c0 * nm + c1 * s
        xt_ref[pl.ds(t * 8, 8), :] = nm
        return nm

    nm_fin = lax.fori_loop(0, tt, body, nmda_ref[...], unroll=16)
    nmda_ref[...] = nm_fin

    r = tt * 8
    x = xt_ref[...]
    rmax = jnp.max(x, axis=1, keepdims=True)
    rm1 = wta * rmax
    scores = x * wta
    lane_r = lax.broadcasted_iota(jnp.int32, (r, w), 1)
    l1 = jnp.min(jnp.where(scores == rm1, lane_r, big), axis=1, keepdims=True)
    s2 = jnp.where(lane_r != l1, scores, neg)
    rm2 = jnp.max(s2, axis=1, keepdims=True)
    l2 = jnp.min(jnp.where(s2 == rm2, lane_r, big), axis=1, keepdims=True)

    sub = lax.broadcasted_iota(jnp.int32, (r, 1), 0) % 8
    base_r = pl.program_id(0) * (8 * w) + sub * w
    rmax_ref[0] = rmax
    rm2_ref[0] = rm2
    gi1_ref[0] = base_r + l1
    gi2_ref[0] = base_r + l2


def _stage2_body(rmax_ref, rm2_ref, gi1_ref, gi2_ref,
                 mask_ref, cov_ref, *, b2, cov_c):
    jb = pl.program_id(1)
    big = jnp.int32(_BIG)
    wta = jnp.float32(_WTA_INH)
    neg = jnp.float32(-jnp.inf)
    rmax = rmax_ref[...]
    v2 = rm2_ref[...]
    gi1 = gi1_ref[...]
    gi2 = gi2_ref[...]

    nmaxg = jnp.max(rmax, axis=1, keepdims=True)
    ign = nmaxg >= jnp.float32(_IGNITE_THR)

    v1 = rmax * wta
    gm1 = nmaxg * wta
    iw1 = jnp.min(jnp.where(v1 == gm1, gi1, big), axis=1, keepdims=True)

    v1x = jnp.where(gi1 == iw1, neg, v1)
    gm2 = jnp.maximum(jnp.max(v1x, axis=1, keepdims=True),
                      jnp.max(v2, axis=1, keepdims=True))
    c1b = jnp.min(jnp.where(v1x == gm2, gi1, big), axis=1, keepdims=True)
    c2b = jnp.min(jnp.where(v2 == gm2, gi2, big), axis=1, keepdims=True)
    iw2 = jnp.minimum(c1b, c2b)

    iw1m = jnp.where(ign, iw1, -1)
    iw2m = jnp.where(ign, iw2, -1)

    tt = rmax.shape[0]
    lanei = lax.broadcasted_iota(jnp.int32, (tt, b2), 1) + jb * b2
    m = (lanei == iw1m) | (lanei == iw2m)
    mask_ref[...] = m.astype(jnp.float32)

    @pl.when(jb == 0)
    def _():
        cov_ref[...] = jnp.where(ign, jnp.float32(cov_c), jnp.float32(0.0))


def kernel(spikes, nmda_state):
    t_dim, n_dim = spikes.shape
    b = 4096 if n_dim % 4096 == 0 else n_dim
    nb = n_dim // b
    w = b // 8
    nc = nb * 8
    tt = 512 if t_dim % 512 == 0 else t_dim
    nt = t_dim // tt

    n0 = nmda_state.reshape(nb, 8, w)

    cand_f = jax.ShapeDtypeStruct((nb, 8 * t_dim, 1), jnp.float32)
    cand_i = jax.ShapeDtypeStruct((nb, 8 * t_dim, 1), jnp.int32)

    s1 = pl.pallas_call(
        functools.partial(_stage1_body, tt=tt, w=w, b=b, nb=nb, nt=nt),
        grid=(nb, nt),
        in_specs=[
            pl.BlockSpec(memory_space=pl.ANY),
            pl.BlockSpec((1, 8, w), lambda bi, ti: (bi, 0, 0)),
        ],
        out_specs=[pl.BlockSpec((1, 8 * tt, 1), lambda bi, ti: (bi, ti, 0))] * 4,
        out_shape=[cand_f, cand_f, cand_i, cand_i],
        scratch_shapes=[
            pltpu.VMEM((8, w), jnp.float32),
            pltpu.VMEM((8 * tt, w), jnp.float32),
            pltpu.VMEM((2, tt, 8, w), jnp.float32),
            pltpu.SemaphoreType.DMA((2, 8)),
        ],
    )
    rmax, rm2, gi1, gi2 = s1(spikes, n0)

    def to_tc(x):
        return x.reshape(nb, t_dim, 8).transpose(1, 0, 2).reshape(t_dim, nc)

    rmax_t, rm2_t, gi1_t, gi2_t = map(to_tc, (rmax, rm2, gi1, gi2))

    b2 = 4096 if n_dim % 4096 == 0 else n_dim
    nb2 = n_dim // b2
    cov_c = float(np.float32(2.0) / np.float32(n_dim))

    s2 = pl.pallas_call(
        functools.partial(_stage2_body, b2=b2, cov_c=cov_c),
        grid=(nt, nb2),
        in_specs=[pl.BlockSpec((tt, nc), lambda ti, jb: (ti, 0))] * 4,
        out_specs=[
            pl.BlockSpec((tt, b2), lambda ti, jb: (ti, jb)),
            pl.BlockSpec((tt, 1), lambda ti, jb: (ti, 0)),
        ],
        out_shape=[
            jax.ShapeDtypeStruct((t_dim, n_dim), jnp.float32),
            jax.ShapeDtypeStruct((t_dim, 1), jnp.float32),
        ],
    )
    mask, cov = s2(rmax_t, rm2_t, gi1_t, gi2_t)
    return mask, cov.reshape(t_dim)
